# baseline (speedup 1.0000x reference)
import functools

import jax
import jax.numpy as jnp
from jax import lax
from jax.experimental import pallas as pl
from jax.experimental.pallas import tpu as pltpu
from jax.experimental.pallas import tpu_sc as plsc

VOCAB = 1000000
EMB = 32
BATCH = 4096
SEQ = 200
NW = 32
BBLK = 128
NR = 4
AHEAD = 3
NSTG = 2
UNROLL = 8
L = 16


def _make_kernel():
    mesh = plsc.VectorSubcoreMesh(core_axis_name="c", subcore_axis_name="s")

    @functools.partial(
        pl.kernel,
        out_type=(
            jax.ShapeDtypeStruct((SEQ, EMB, BATCH), jnp.float32),
            jax.ShapeDtypeStruct((SEQ, BATCH), jnp.float32),
        ),
        mesh=mesh,
        compiler_params=pltpu.CompilerParams(
            use_tc_tiling_on_sc=True, needs_layout_passes=False),
        scratch_types=(
            [
                pltpu.VMEM((SEQ, BBLK), jnp.int32),
                pltpu.VMEM((NR, BBLK), jnp.int32),
                pltpu.VMEM((NR, BBLK, BBLK), jnp.float32),
                pltpu.VMEM((NSTG, EMB, BBLK), jnp.float32),
                pltpu.VMEM((8, BBLK), jnp.float32),
            ]
            + [pltpu.SemaphoreType.DMA] * NR
            + [pltpu.SemaphoreType.DMA] * NSTG
            + [pltpu.SemaphoreType.DMA]
        ),
    )
    def emb_kernel(xt_hbm, w5_hbm, p_hbm, maskt_hbm,
                   idx_v, idx4_v, rows_v, stage_v, mask_v, *sems):
        gsems = sems[:NR]
        osems = sems[NR:NR + NSTG]
        wid = lax.axis_index("s") * 2 + lax.axis_index("c")
        col0 = wid * BBLK

        pltpu.sync_copy(xt_hbm.at[:, pl.ds(col0, BBLK)], idx_v)

        iota = lax.iota(jnp.int32, L)

        def prep_gather(g, slot):
            for jb in range(BBLK // L):
                v = idx_v[g, pl.ds(jb * L, L)]
                idx4_v[slot, pl.ds(jb * L, L)] = lax.shift_right_logical(v, 2)
            pltpu.async_copy(
                w5_hbm.at[idx4_v.at[slot]], rows_v.at[slot], gsems[slot])

        def visit(t, u):
            tm8 = u % 8
            stg = u % NSTG
            slot = u % NR
            slot2 = (u + AHEAD) % NR

            g2 = t + AHEAD

            @pl.when(g2 < SEQ)
            def _():
                prep_gather(g2, slot2)

            for jb in range(BBLK // L):
                v = idx_v[t, pl.ds(jb * L, L)]
                mask_v[tm8, pl.ds(jb * L, L)] = jnp.where(
                    v != 0, jnp.float32(1.0), jnp.float32(0.0))

            pltpu.make_async_copy(
                w5_hbm.at[idx4_v.at[slot]], rows_v.at[slot],
                gsems[slot]).wait()

            @pl.when(t >= NSTG)
            def _():
                pltpu.make_async_copy(
                    stage_v.at[stg],
                    p_hbm.at[t - NSTG, :, pl.ds(col0, BBLK)],
                    osems[stg]).wait()

            exts = []
            rowids = []
            for jb in range(BBLK // L):
                v = idx_v[t, pl.ds(jb * L, L)]
                exts.append(lax.shift_left(jnp.bitwise_and(v, 3), 5))
                rowids.append(iota + (jb * L))

            def e_body(e, _):
                for jb in range(BBLK // L):
                    vals = plsc.load_gather(
                        rows_v.at[slot], [rowids[jb], exts[jb] + e])
                    stage_v[stg, e, pl.ds(jb * L, L)] = vals
                return 0

            lax.fori_loop(0, EMB, e_body, 0)

            pltpu.async_copy(
                stage_v.at[stg], p_hbm.at[t, :, pl.ds(col0, BBLK)],
                osems[stg])

            if tm8 == 7:
                pltpu.sync_copy(
                    mask_v, maskt_hbm.at[pl.ds(t - 7, 8), pl.ds(col0, BBLK)])

        for g in range(AHEAD):
            prep_gather(g, g % NR)

        def step(s, _):
            base = s * UNROLL
            for u in range(UNROLL):
                visit(base + u, u)
            return 0

        lax.fori_loop(0, SEQ // UNROLL, step, 0, unroll=False)

        for u in range(NSTG):
            t = SEQ - NSTG + u
            pltpu.make_async_copy(
                stage_v.at[t % NSTG],
                p_hbm.at[t, :, pl.ds(col0, BBLK)],
                osems[t % NSTG]).wait()

    return emb_kernel


_emb_kernel = None


def kernel(x, W):
    global _emb_kernel
    if _emb_kernel is None:
        _emb_kernel = _make_kernel()
    xt = x.T.astype(jnp.int32)
    w5 = W.reshape(VOCAB // 4, EMB * 4)
    p, maskt = _emb_kernel(xt, w5)
    return p.transpose(2, 0, 1), maskt.T
